# baseline (speedup 1.0000x reference)
import jax
import jax.numpy as jnp
from jax.experimental import pallas as pl

NUM_CLASSES = 80
NUM_TOP = 300
NUM_KPTS = 9


def _post_body(idx_ref, val_ref, box_ref, kpt_ref, vis_ref, sz_ref,
               lab_ref, boxo_ref, sco_ref, kpto_ref, viso_ref):
    Q = box_ref.shape[1]
    C = NUM_CLASSES
    idx = idx_ref[0]
    qidx = idx // C
    lab_ref[0] = idx - qidx * C + 1
    sco_ref[0] = jax.nn.sigmoid(val_ref[0])

    qcol = jnp.transpose(qidx)
    iota = jax.lax.broadcasted_iota(jnp.int32, (NUM_TOP, Q), 1)
    onehot = jnp.where(iota == qcol, 1.0, 0.0).astype(jnp.float32)
    data = jnp.concatenate([box_ref[0], kpt_ref[0], vis_ref[0]], axis=-1)
    g = jnp.dot(onehot, data, preferred_element_type=jnp.float32)

    w_img = sz_ref[0, 0, 0]
    h_img = sz_ref[0, 0, 1]
    cx = g[:, 0:1]
    cy = g[:, 1:2]
    w = g[:, 2:3]
    h = g[:, 3:4]
    boxo_ref[0] = jnp.concatenate(
        [(cx - 0.5 * w) * w_img, (cy - 0.5 * h) * h_img,
         (cx + 0.5 * w) * w_img, (cy + 0.5 * h) * h_img], axis=-1)
    scale_xy = jnp.where(
        jax.lax.broadcasted_iota(jnp.int32, (1, 2 * NUM_KPTS), 1) % 2 == 0,
        w_img, h_img)
    kpto_ref[0] = g[:, 4:4 + 2 * NUM_KPTS] * scale_xy
    viso_ref[0] = jax.nn.sigmoid(g[:, 4 + 2 * NUM_KPTS:4 + 3 * NUM_KPTS])


def kernel(pred_logits, pred_boxes, pred_keypoints, pred_kpt_vis, orig_target_sizes):
    B, Q, C = pred_logits.shape
    K = NUM_KPTS
    top_vals, index = jax.lax.top_k(pred_logits.reshape(B, Q * C), NUM_TOP)
    index = index.reshape(B, 1, NUM_TOP)
    top_vals = top_vals.reshape(B, 1, NUM_TOP)
    sizes3 = orig_target_sizes.reshape(B, 1, 2)

    grid = (B,)
    in_specs = [
        pl.BlockSpec((1, 1, NUM_TOP), lambda b: (b, 0, 0)),
        pl.BlockSpec((1, 1, NUM_TOP), lambda b: (b, 0, 0)),
        pl.BlockSpec((1, Q, 4), lambda b: (b, 0, 0)),
        pl.BlockSpec((1, Q, 2 * K), lambda b: (b, 0, 0)),
        pl.BlockSpec((1, Q, K), lambda b: (b, 0, 0)),
        pl.BlockSpec((1, 1, 2), lambda b: (b, 0, 0)),
    ]
    out_specs = [
        pl.BlockSpec((1, 1, NUM_TOP), lambda b: (b, 0, 0)),
        pl.BlockSpec((1, NUM_TOP, 4), lambda b: (b, 0, 0)),
        pl.BlockSpec((1, 1, NUM_TOP), lambda b: (b, 0, 0)),
        pl.BlockSpec((1, NUM_TOP, 2 * K), lambda b: (b, 0, 0)),
        pl.BlockSpec((1, NUM_TOP, K), lambda b: (b, 0, 0)),
    ]
    out_shape = [
        jax.ShapeDtypeStruct((B, 1, NUM_TOP), jnp.int32),
        jax.ShapeDtypeStruct((B, NUM_TOP, 4), jnp.float32),
        jax.ShapeDtypeStruct((B, 1, NUM_TOP), jnp.float32),
        jax.ShapeDtypeStruct((B, NUM_TOP, 2 * K), jnp.float32),
        jax.ShapeDtypeStruct((B, NUM_TOP, K), jnp.float32),
    ]
    labels, boxes_out, top_scores, kpts_flat, vis_out = pl.pallas_call(
        _post_body,
        grid=grid,
        in_specs=in_specs,
        out_specs=out_specs,
        out_shape=out_shape,
    )(index, top_vals, pred_boxes, pred_keypoints, pred_kpt_vis, sizes3)
    kpts_out = kpts_flat.reshape(B, NUM_TOP, K, 2)
    return (labels.reshape(B, NUM_TOP), boxes_out, top_scores.reshape(B, NUM_TOP), kpts_out, vis_out)
